# baseline (speedup 1.0000x reference)
import functools

import jax
import jax.numpy as jnp
from jax import lax
from jax.experimental import pallas as pl
from jax.experimental.pallas import tpu as pltpu
from jax.experimental.pallas import tpu_sc as plsc

N_NODES = 10000
D_FEAT = 128
N_EDGES = 320000

NUM_CORES = 2
NUM_SUBCORES = 16
NUM_WORKERS = NUM_CORES * NUM_SUBCORES

CHUNK = 128
FAST_C = 0
NCK_F = 102
NCK_S = 55
NCK_MAX = max(NCK_F, NCK_S)
NCK_BAL = 79
PHASES = 2
PC = -(-NCK_MAX // PHASES)
E_PAD = (NCK_F + NCK_S) * CHUNK * NUM_SUBCORES

TRASH = N_NODES
ACC_ROWS = 10112
ZROWS = ACC_ROWS // NUM_SUBCORES

ROW_BLOCK = 400
DEG_LANES = 128

_MESH = plsc.VectorSubcoreMesh(core_axis_name="c", subcore_axis_name="s")


def _my_nck(c):
    return jnp.where(c == FAST_C, NCK_F, NCK_S)


@functools.partial(
    pl.kernel, mesh=_MESH,
    out_type=jax.ShapeDtypeStruct((NUM_CORES, ACC_ROWS, D_FEAT), jnp.float32),
    scratch_types=[
        pltpu.VMEM((PC, CHUNK), jnp.int32),
        pltpu.VMEM((PC, CHUNK), jnp.int32),
        pltpu.VMEM((CHUNK, D_FEAT), jnp.float32),
        pltpu.VMEM_SHARED((ACC_ROWS, D_FEAT), jnp.float32),
        pltpu.SemaphoreType.DMA,
    ])
def _sc_agg(h_hbm, srci_hbm, dsti_hbm, zeros_hbm, outa_hbm,
            src_v, dst_v, rows_v, acc_sh, sem):
    c = lax.axis_index("c")
    s = lax.axis_index("s")
    wid = s * NUM_CORES + c
    nck = _my_nck(c)

    pltpu.sync_copy(zeros_hbm, acc_sh.at[pl.ds(s * ZROWS, ZROWS)])
    plsc.subcore_barrier()

    def body(j, carry):
        pltpu.async_copy(h_hbm.at[src_v.at[j]], rows_v, sem).wait()
        pltpu.sync_copy(rows_v, acc_sh.at[dst_v.at[j]], add=True)
        return carry

    for p in range(PHASES):
        pltpu.sync_copy(srci_hbm.at[wid, p], src_v)
        pltpu.sync_copy(dsti_hbm.at[wid, p], dst_v)
        trip = jnp.clip(nck - p * PC, 0, PC)
        lax.fori_loop(0, trip, body, 0)

    plsc.subcore_barrier()
    r0 = s * ZROWS
    pltpu.sync_copy(acc_sh.at[pl.ds(r0, ZROWS)],
                    outa_hbm.at[c, pl.ds(r0, ZROWS)])


@functools.partial(
    pl.kernel, mesh=_MESH,
    out_type=jax.ShapeDtypeStruct((NUM_CORES, ACC_ROWS, DEG_LANES),
                                  jnp.float32),
    scratch_types=[
        pltpu.VMEM((PC, CHUNK), jnp.int32),
        pltpu.VMEM((CHUNK, DEG_LANES), jnp.float32),
        pltpu.VMEM_SHARED((ACC_ROWS, DEG_LANES), jnp.float32),
        pltpu.SemaphoreType.DMA,
    ])
def _sc_degree(dsti_hbm, zerosd_hbm, ones_hbm, outd_hbm,
               dst_v, ones_v, deg_sh, sem):
    c = lax.axis_index("c")
    s = lax.axis_index("s")
    wid = s * NUM_CORES + c
    nck = NCK_BAL

    pltpu.sync_copy(zerosd_hbm, deg_sh.at[pl.ds(s * ZROWS, ZROWS)])
    pltpu.sync_copy(ones_hbm, ones_v)
    plsc.subcore_barrier()

    def body(j, carry):
        pltpu.sync_copy(ones_v, deg_sh.at[dst_v.at[j]], add=True)
        return carry

    for p in range(PHASES):
        pltpu.sync_copy(dsti_hbm.at[wid, p], dst_v)
        trip = jnp.clip(nck - p * PC, 0, PC)
        lax.fori_loop(0, trip, body, 0)

    plsc.subcore_barrier()
    r0 = s * ZROWS
    pltpu.sync_copy(deg_sh.at[pl.ds(r0, ZROWS)],
                    outd_hbm.at[c, pl.ds(r0, ZROWS)])


def _tc_layer_body(ap_ref, h_ref, degp_ref, w_ref, b_ref, o_ref, *, act):
    a = ap_ref[0] + ap_ref[1] + h_ref[...]
    deg = degp_ref[0, :, 0:1] + degp_ref[1, :, 0:1] + 1.0
    y = jnp.dot(a, w_ref[...], preferred_element_type=jnp.float32)
    y = y / deg + b_ref[...]
    o_ref[...] = act(y)


def _make_tc_layer(out_c, act):
    grid = (N_NODES // ROW_BLOCK,)
    return pl.pallas_call(
        functools.partial(_tc_layer_body, act=act),
        grid=grid,
        in_specs=[
            pl.BlockSpec((NUM_CORES, ROW_BLOCK, D_FEAT), lambda i: (0, i, 0)),
            pl.BlockSpec((ROW_BLOCK, D_FEAT), lambda i: (i, 0)),
            pl.BlockSpec((NUM_CORES, ROW_BLOCK, 8), lambda i: (0, i, 0)),
            pl.BlockSpec((D_FEAT, out_c), lambda i: (0, 0)),
            pl.BlockSpec((1, out_c), lambda i: (0, 0)),
        ],
        out_specs=pl.BlockSpec((ROW_BLOCK, out_c), lambda i: (i, 0)),
        out_shape=jax.ShapeDtypeStruct((N_NODES, out_c), jnp.float32),
    )


_relu = lambda y: jnp.maximum(y, 0.0)
_tc_hidden = _make_tc_layer(D_FEAT, _relu)
_tc_final = _make_tc_layer(64, jnp.tanh)


def _build_worker_indices(src, dstp, nck_f, nck_s):
    e_pad = (nck_f + nck_s) * CHUNK * NUM_SUBCORES
    pad = e_pad - N_EDGES
    srci = jnp.concatenate([src, jnp.zeros((pad,), jnp.int32)])
    dsti = jnp.concatenate([dstp, jnp.full((pad,), TRASH, jnp.int32)])
    src_rows, dst_rows = [], []
    off = 0
    fill = PHASES * PC * CHUNK
    for wid in range(NUM_WORKERS):
        n = (nck_f if (wid % NUM_CORES) == FAST_C else nck_s) * CHUNK
        s_blk = srci[off:off + n]
        d_blk = dsti[off:off + n]
        if n < fill:
            s_blk = jnp.concatenate(
                [s_blk, jnp.zeros((fill - n,), jnp.int32)])
            d_blk = jnp.concatenate(
                [d_blk, jnp.full((fill - n,), TRASH, jnp.int32)])
        src_rows.append(s_blk.reshape(PHASES, PC, CHUNK))
        dst_rows.append(d_blk.reshape(PHASES, PC, CHUNK))
        off += n
    return jnp.stack(src_rows), jnp.stack(dst_rows)


def kernel(x, W_in, U_in, C_in, B_in, W_h0, U_h0, C_h0, B_h0,
           W_h1, U_h1, C_h1, B_h1, W_out, U_out, C_out, B_out, edge_index):
    src = edge_index[0].astype(jnp.int32)
    dst = edge_index[1].astype(jnp.int32)
    dstp = jnp.where(src == dst, TRASH, dst)
    srci, dsti = _build_worker_indices(src, dstp, NCK_F, NCK_S)
    _, dsti_bal = _build_worker_indices(src, dstp, NCK_BAL, NCK_BAL)

    zeros_h = jnp.zeros((ZROWS, D_FEAT), jnp.float32)
    zerosd_h = jnp.zeros((ZROWS, DEG_LANES), jnp.float32)
    ones_h = jnp.ones((CHUNK, DEG_LANES), jnp.float32)

    degp = _sc_degree(dsti_bal, zerosd_h, ones_h)[:, :, :8]
    ap = _sc_agg(x, srci, dsti, zeros_h)
    h = _tc_hidden(ap, x, degp, W_in, B_in.reshape(1, -1))
    ap = _sc_agg(h, srci, dsti, zeros_h)
    h = _tc_hidden(ap, h, degp, W_h0, B_h0.reshape(1, -1))
    ap = _sc_agg(h, srci, dsti, zeros_h)
    h = _tc_hidden(ap, h, degp, W_h1, B_h1.reshape(1, -1))
    ap = _sc_agg(h, srci, dsti, zeros_h)
    return _tc_final(ap, h, degp, W_out, B_out.reshape(1, -1))


# speedup vs baseline: 1.0777x; 1.0777x over previous
import functools

import jax
import jax.numpy as jnp
from jax import lax
from jax.experimental import pallas as pl
from jax.experimental.pallas import tpu as pltpu
from jax.experimental.pallas import tpu_sc as plsc

N_NODES = 10000
D_FEAT = 128
N_EDGES = 320000

NUM_CORES = 2
NUM_SUBCORES = 16
NUM_WORKERS = NUM_CORES * NUM_SUBCORES

CHUNK = 128
FAST_C = 0
NCK_F = 95
NCK_S = 62
NCK_MAX = max(NCK_F, NCK_S)
NCK_BAL = 79
PHASES = 2
PC = -(-NCK_MAX // PHASES)
E_PAD = (NCK_F + NCK_S) * CHUNK * NUM_SUBCORES

TRASH = N_NODES
ACC_ROWS = 10112
ZROWS = ACC_ROWS // NUM_SUBCORES

ROW_BLOCK = 400
DEG_LANES = 128

_MESH = plsc.VectorSubcoreMesh(core_axis_name="c", subcore_axis_name="s")


def _my_nck(c):
    return jnp.where(c == FAST_C, NCK_F, NCK_S)


@functools.partial(
    pl.kernel, mesh=_MESH,
    out_type=jax.ShapeDtypeStruct((NUM_CORES, ACC_ROWS, D_FEAT), jnp.float32),
    scratch_types=[
        pltpu.VMEM((PC, CHUNK), jnp.int32),
        pltpu.VMEM((PC, CHUNK), jnp.int32),
        pltpu.VMEM((CHUNK, D_FEAT), jnp.float32),
        pltpu.VMEM_SHARED((ACC_ROWS, D_FEAT), jnp.float32),
        pltpu.SemaphoreType.DMA,
    ])
def _sc_agg(h_hbm, srci_hbm, dsti_hbm, zeros_hbm, outa_hbm,
            src_v, dst_v, rows_v, acc_sh, sem):
    c = lax.axis_index("c")
    s = lax.axis_index("s")
    wid = s * NUM_CORES + c
    nck = _my_nck(c)

    pltpu.sync_copy(zeros_hbm, acc_sh.at[pl.ds(s * ZROWS, ZROWS)])
    plsc.subcore_barrier()

    def body(j, carry):
        pltpu.async_copy(h_hbm.at[src_v.at[j]], rows_v, sem).wait()
        pltpu.sync_copy(rows_v, acc_sh.at[dst_v.at[j]], add=True)
        return carry

    for p in range(PHASES):
        pltpu.sync_copy(srci_hbm.at[wid, p], src_v)
        pltpu.sync_copy(dsti_hbm.at[wid, p], dst_v)
        trip = jnp.clip(nck - p * PC, 0, PC)
        lax.fori_loop(0, trip, body, 0)

    plsc.subcore_barrier()
    r0 = s * ZROWS
    pltpu.sync_copy(acc_sh.at[pl.ds(r0, ZROWS)],
                    outa_hbm.at[c, pl.ds(r0, ZROWS)])


@functools.partial(
    pl.kernel, mesh=_MESH,
    out_type=jax.ShapeDtypeStruct((NUM_CORES, ACC_ROWS, DEG_LANES),
                                  jnp.float32),
    scratch_types=[
        pltpu.VMEM((PC, CHUNK), jnp.int32),
        pltpu.VMEM((CHUNK, DEG_LANES), jnp.float32),
        pltpu.VMEM_SHARED((ACC_ROWS, DEG_LANES), jnp.float32),
        pltpu.SemaphoreType.DMA,
    ])
def _sc_degree(dsti_hbm, zerosd_hbm, ones_hbm, outd_hbm,
               dst_v, ones_v, deg_sh, sem):
    c = lax.axis_index("c")
    s = lax.axis_index("s")
    wid = s * NUM_CORES + c
    nck = NCK_BAL

    pltpu.sync_copy(zerosd_hbm, deg_sh.at[pl.ds(s * ZROWS, ZROWS)])
    pltpu.sync_copy(ones_hbm, ones_v)
    plsc.subcore_barrier()

    def body(j, carry):
        pltpu.sync_copy(ones_v, deg_sh.at[dst_v.at[j]], add=True)
        return carry

    for p in range(PHASES):
        pltpu.sync_copy(dsti_hbm.at[wid, p], dst_v)
        trip = jnp.clip(nck - p * PC, 0, PC)
        lax.fori_loop(0, trip, body, 0)

    plsc.subcore_barrier()
    r0 = s * ZROWS
    pltpu.sync_copy(deg_sh.at[pl.ds(r0, ZROWS)],
                    outd_hbm.at[c, pl.ds(r0, ZROWS)])


def _tc_layer_body(ap_ref, h_ref, degp_ref, w_ref, b_ref, o_ref, *, act):
    a = ap_ref[0] + ap_ref[1] + h_ref[...]
    deg = degp_ref[0, :, 0:1] + degp_ref[1, :, 0:1] + 1.0
    y = jnp.dot(a, w_ref[...], preferred_element_type=jnp.float32)
    y = y / deg + b_ref[...]
    o_ref[...] = act(y)


def _make_tc_layer(out_c, act):
    grid = (N_NODES // ROW_BLOCK,)
    return pl.pallas_call(
        functools.partial(_tc_layer_body, act=act),
        grid=grid,
        in_specs=[
            pl.BlockSpec((NUM_CORES, ROW_BLOCK, D_FEAT), lambda i: (0, i, 0)),
            pl.BlockSpec((ROW_BLOCK, D_FEAT), lambda i: (i, 0)),
            pl.BlockSpec((NUM_CORES, ROW_BLOCK, 8), lambda i: (0, i, 0)),
            pl.BlockSpec((D_FEAT, out_c), lambda i: (0, 0)),
            pl.BlockSpec((1, out_c), lambda i: (0, 0)),
        ],
        out_specs=pl.BlockSpec((ROW_BLOCK, out_c), lambda i: (i, 0)),
        out_shape=jax.ShapeDtypeStruct((N_NODES, out_c), jnp.float32),
    )


_relu = lambda y: jnp.maximum(y, 0.0)
_tc_hidden = _make_tc_layer(D_FEAT, _relu)
_tc_final = _make_tc_layer(64, jnp.tanh)


def _build_worker_indices(src, dstp, nck_f, nck_s):
    e_pad = (nck_f + nck_s) * CHUNK * NUM_SUBCORES
    pad = e_pad - N_EDGES
    srci = jnp.concatenate([src, jnp.zeros((pad,), jnp.int32)])
    dsti = jnp.concatenate([dstp, jnp.full((pad,), TRASH, jnp.int32)])
    src_rows, dst_rows = [], []
    off = 0
    fill = PHASES * PC * CHUNK
    for wid in range(NUM_WORKERS):
        n = (nck_f if (wid % NUM_CORES) == FAST_C else nck_s) * CHUNK
        s_blk = srci[off:off + n]
        d_blk = dsti[off:off + n]
        if n < fill:
            s_blk = jnp.concatenate(
                [s_blk, jnp.zeros((fill - n,), jnp.int32)])
            d_blk = jnp.concatenate(
                [d_blk, jnp.full((fill - n,), TRASH, jnp.int32)])
        src_rows.append(s_blk.reshape(PHASES, PC, CHUNK))
        dst_rows.append(d_blk.reshape(PHASES, PC, CHUNK))
        off += n
    return jnp.stack(src_rows), jnp.stack(dst_rows)


def kernel(x, W_in, U_in, C_in, B_in, W_h0, U_h0, C_h0, B_h0,
           W_h1, U_h1, C_h1, B_h1, W_out, U_out, C_out, B_out, edge_index):
    src = edge_index[0].astype(jnp.int32)
    dst = edge_index[1].astype(jnp.int32)
    dstp = jnp.where(src == dst, TRASH, dst)
    srci, dsti = _build_worker_indices(src, dstp, NCK_F, NCK_S)
    _, dsti_bal = _build_worker_indices(src, dstp, NCK_BAL, NCK_BAL)

    zeros_h = jnp.zeros((ZROWS, D_FEAT), jnp.float32)
    zerosd_h = jnp.zeros((ZROWS, DEG_LANES), jnp.float32)
    ones_h = jnp.ones((CHUNK, DEG_LANES), jnp.float32)

    degp = _sc_degree(dsti_bal, zerosd_h, ones_h)[:, :, :8]
    ap = _sc_agg(x, srci, dsti, zeros_h)
    h = _tc_hidden(ap, x, degp, W_in, B_in.reshape(1, -1))
    ap = _sc_agg(h, srci, dsti, zeros_h)
    h = _tc_hidden(ap, h, degp, W_h0, B_h0.reshape(1, -1))
    ap = _sc_agg(h, srci, dsti, zeros_h)
    h = _tc_hidden(ap, h, degp, W_h1, B_h1.reshape(1, -1))
    ap = _sc_agg(h, srci, dsti, zeros_h)
    return _tc_final(ap, h, degp, W_out, B_out.reshape(1, -1))


# speedup vs baseline: 1.1040x; 1.0244x over previous
import functools

import jax
import jax.numpy as jnp
from jax import lax
from jax.experimental import pallas as pl
from jax.experimental.pallas import tpu as pltpu
from jax.experimental.pallas import tpu_sc as plsc

N_NODES = 10000
D_FEAT = 128
N_EDGES = 320000

NUM_CORES = 2
NUM_SUBCORES = 16
NUM_WORKERS = NUM_CORES * NUM_SUBCORES

CHUNK = 128
FAST_C = 0
NCK_F = 91
NCK_S = 66
NCK_MAX = max(NCK_F, NCK_S)
NCK_BAL = 79
PHASES = 2
PC = -(-NCK_MAX // PHASES)
E_PAD = (NCK_F + NCK_S) * CHUNK * NUM_SUBCORES

TRASH = N_NODES
ACC_ROWS = 10112
ZROWS = ACC_ROWS // NUM_SUBCORES

ROW_BLOCK = 400
DEG_LANES = 128

_MESH = plsc.VectorSubcoreMesh(core_axis_name="c", subcore_axis_name="s")


def _my_nck(c):
    return jnp.where(c == FAST_C, NCK_F, NCK_S)


@functools.partial(
    pl.kernel, mesh=_MESH,
    out_type=jax.ShapeDtypeStruct((NUM_CORES, ACC_ROWS, D_FEAT), jnp.float32),
    scratch_types=[
        pltpu.VMEM((PC, CHUNK), jnp.int32),
        pltpu.VMEM((PC, CHUNK), jnp.int32),
        pltpu.VMEM((CHUNK, D_FEAT), jnp.float32),
        pltpu.VMEM_SHARED((ACC_ROWS, D_FEAT), jnp.float32),
        pltpu.SemaphoreType.DMA,
    ])
def _sc_agg(h_hbm, srci_hbm, dsti_hbm, zeros_hbm, outa_hbm,
            src_v, dst_v, rows_v, acc_sh, sem):
    c = lax.axis_index("c")
    s = lax.axis_index("s")
    wid = s * NUM_CORES + c
    nck = _my_nck(c)

    pltpu.sync_copy(zeros_hbm, acc_sh.at[pl.ds(s * ZROWS, ZROWS)])
    plsc.subcore_barrier()

    def body(j, carry):
        pltpu.async_copy(h_hbm.at[src_v.at[j]], rows_v, sem).wait()
        pltpu.sync_copy(rows_v, acc_sh.at[dst_v.at[j]], add=True)
        return carry

    for p in range(PHASES):
        pltpu.sync_copy(srci_hbm.at[wid, p], src_v)
        pltpu.sync_copy(dsti_hbm.at[wid, p], dst_v)
        trip = jnp.clip(nck - p * PC, 0, PC)
        lax.fori_loop(0, trip, body, 0)

    plsc.subcore_barrier()
    r0 = s * ZROWS
    pltpu.sync_copy(acc_sh.at[pl.ds(r0, ZROWS)],
                    outa_hbm.at[c, pl.ds(r0, ZROWS)])


@functools.partial(
    pl.kernel, mesh=_MESH,
    out_type=jax.ShapeDtypeStruct((NUM_CORES, ACC_ROWS, DEG_LANES),
                                  jnp.float32),
    scratch_types=[
        pltpu.VMEM((PC, CHUNK), jnp.int32),
        pltpu.VMEM((CHUNK, DEG_LANES), jnp.float32),
        pltpu.VMEM_SHARED((ACC_ROWS, DEG_LANES), jnp.float32),
        pltpu.SemaphoreType.DMA,
    ])
def _sc_degree(dsti_hbm, zerosd_hbm, ones_hbm, outd_hbm,
               dst_v, ones_v, deg_sh, sem):
    c = lax.axis_index("c")
    s = lax.axis_index("s")
    wid = s * NUM_CORES + c
    nck = NCK_BAL

    pltpu.sync_copy(zerosd_hbm, deg_sh.at[pl.ds(s * ZROWS, ZROWS)])
    pltpu.sync_copy(ones_hbm, ones_v)
    plsc.subcore_barrier()

    def body(j, carry):
        pltpu.sync_copy(ones_v, deg_sh.at[dst_v.at[j]], add=True)
        return carry

    for p in range(PHASES):
        pltpu.sync_copy(dsti_hbm.at[wid, p], dst_v)
        trip = jnp.clip(nck - p * PC, 0, PC)
        lax.fori_loop(0, trip, body, 0)

    plsc.subcore_barrier()
    r0 = s * ZROWS
    pltpu.sync_copy(deg_sh.at[pl.ds(r0, ZROWS)],
                    outd_hbm.at[c, pl.ds(r0, ZROWS)])


def _tc_layer_body(ap_ref, h_ref, degp_ref, w_ref, b_ref, o_ref, *, act):
    a = ap_ref[0] + ap_ref[1] + h_ref[...]
    deg = degp_ref[0, :, 0:1] + degp_ref[1, :, 0:1] + 1.0
    y = jnp.dot(a, w_ref[...], preferred_element_type=jnp.float32)
    y = y / deg + b_ref[...]
    o_ref[...] = act(y)


def _make_tc_layer(out_c, act):
    grid = (N_NODES // ROW_BLOCK,)
    return pl.pallas_call(
        functools.partial(_tc_layer_body, act=act),
        grid=grid,
        in_specs=[
            pl.BlockSpec((NUM_CORES, ROW_BLOCK, D_FEAT), lambda i: (0, i, 0)),
            pl.BlockSpec((ROW_BLOCK, D_FEAT), lambda i: (i, 0)),
            pl.BlockSpec((NUM_CORES, ROW_BLOCK, 8), lambda i: (0, i, 0)),
            pl.BlockSpec((D_FEAT, out_c), lambda i: (0, 0)),
            pl.BlockSpec((1, out_c), lambda i: (0, 0)),
        ],
        out_specs=pl.BlockSpec((ROW_BLOCK, out_c), lambda i: (i, 0)),
        out_shape=jax.ShapeDtypeStruct((N_NODES, out_c), jnp.float32),
    )


_relu = lambda y: jnp.maximum(y, 0.0)
_tc_hidden = _make_tc_layer(D_FEAT, _relu)
_tc_final = _make_tc_layer(64, jnp.tanh)


def _build_worker_indices(src, dstp, nck_f, nck_s):
    e_pad = (nck_f + nck_s) * CHUNK * NUM_SUBCORES
    pad = e_pad - N_EDGES
    srci = jnp.concatenate([src, jnp.zeros((pad,), jnp.int32)])
    dsti = jnp.concatenate([dstp, jnp.full((pad,), TRASH, jnp.int32)])
    src_rows, dst_rows = [], []
    off = 0
    fill = PHASES * PC * CHUNK
    for wid in range(NUM_WORKERS):
        n = (nck_f if (wid % NUM_CORES) == FAST_C else nck_s) * CHUNK
        s_blk = srci[off:off + n]
        d_blk = dsti[off:off + n]
        if n < fill:
            s_blk = jnp.concatenate(
                [s_blk, jnp.zeros((fill - n,), jnp.int32)])
            d_blk = jnp.concatenate(
                [d_blk, jnp.full((fill - n,), TRASH, jnp.int32)])
        src_rows.append(s_blk.reshape(PHASES, PC, CHUNK))
        dst_rows.append(d_blk.reshape(PHASES, PC, CHUNK))
        off += n
    return jnp.stack(src_rows), jnp.stack(dst_rows)


def kernel(x, W_in, U_in, C_in, B_in, W_h0, U_h0, C_h0, B_h0,
           W_h1, U_h1, C_h1, B_h1, W_out, U_out, C_out, B_out, edge_index):
    src = edge_index[0].astype(jnp.int32)
    dst = edge_index[1].astype(jnp.int32)
    dstp = jnp.where(src == dst, TRASH, dst)
    srci, dsti = _build_worker_indices(src, dstp, NCK_F, NCK_S)
    _, dsti_bal = _build_worker_indices(src, dstp, NCK_BAL, NCK_BAL)

    zeros_h = jnp.zeros((ZROWS, D_FEAT), jnp.float32)
    zerosd_h = jnp.zeros((ZROWS, DEG_LANES), jnp.float32)
    ones_h = jnp.ones((CHUNK, DEG_LANES), jnp.float32)

    degp = _sc_degree(dsti_bal, zerosd_h, ones_h)[:, :, :8]
    ap = _sc_agg(x, srci, dsti, zeros_h)
    h = _tc_hidden(ap, x, degp, W_in, B_in.reshape(1, -1))
    ap = _sc_agg(h, srci, dsti, zeros_h)
    h = _tc_hidden(ap, h, degp, W_h0, B_h0.reshape(1, -1))
    ap = _sc_agg(h, srci, dsti, zeros_h)
    h = _tc_hidden(ap, h, degp, W_h1, B_h1.reshape(1, -1))
    ap = _sc_agg(h, srci, dsti, zeros_h)
    return _tc_final(ap, h, degp, W_out, B_out.reshape(1, -1))
